# baseline (speedup 1.0000x reference)
import functools

import jax
import jax.numpy as jnp
from jax import lax
from jax.experimental import pallas as pl
from jax.experimental.pallas import tpu as pltpu
from jax.experimental.pallas import tpu_sc as plsc

N = 10000
NP = 10240
E = 320000
F_IN = 128
F_OUT = 64

NC = 2
NS = 16
NW = NC * NS
EPT = E // NW
CHUNK = 125
NCHUNK = EPT // CHUNK
G = 10
NG = NCHUNK // G
HB = 64
RPT = NP // NS
FH = 16

_MESH = plsc.VectorSubcoreMesh(core_axis_name="c", subcore_axis_name="s")

BN = 1000
GRID = N // BN


@functools.partial(
    pl.kernel,
    out_type=jax.ShapeDtypeStruct((NC, NP, FH), jnp.float32),
    mesh=_MESH,
    scratch_types=[
        pltpu.VMEM((NCHUNK, CHUNK), jnp.int32),
        pltpu.VMEM((CHUNK, FH), jnp.float32),
        pltpu.VMEM_SHARED((NP, FH), jnp.float32),
    ],
    compiler_params=pltpu.CompilerParams(use_tc_tiling_on_sc=False),
)
def _hist(col_hbm, ones_hbm, zeros_hbm, out_hbm, col_v, ones_v, acc):
    cid = lax.axis_index("c")
    sid = lax.axis_index("s")
    wid = cid * NS + sid
    pltpu.sync_copy(col_hbm.at[wid], col_v)
    pltpu.sync_copy(ones_hbm, ones_v)
    pltpu.sync_copy(zeros_hbm, acc.at[pl.ds(sid * RPT, RPT)])
    plsc.subcore_barrier()

    def body(j, carry):
        pltpu.sync_copy(ones_v, acc.at[col_v.at[j]], add=True)
        return carry

    lax.fori_loop(0, NCHUNK, body, 0)
    plsc.subcore_barrier()
    pltpu.sync_copy(acc.at[pl.ds(sid * RPT, RPT)],
                    out_hbm.at[cid, pl.ds(sid * RPT, RPT)])


@functools.partial(
    pl.kernel,
    out_type=jax.ShapeDtypeStruct((NC, NP, F_IN), jnp.float32),
    mesh=_MESH,
    scratch_types=[
        pltpu.VMEM((G, CHUNK), jnp.int32),
        pltpu.VMEM((G, CHUNK), jnp.int32),
        pltpu.VMEM((CHUNK, F_IN), jnp.float32),
        pltpu.VMEM((CHUNK, F_IN), jnp.float32),
        pltpu.VMEM_SHARED((NP, F_IN), jnp.float32),
        pltpu.SemaphoreType.DMA,
        pltpu.SemaphoreType.DMA,
    ],
    compiler_params=pltpu.CompilerParams(use_tc_tiling_on_sc=False),
)
def _prop(g_hbm, row_hbm, col_hbm, zeros_hbm, out_hbm,
          slabr, slabc, buf0, buf1, acc, sem0, sem1):
    cid = lax.axis_index("c")
    sid = lax.axis_index("s")
    wid = cid * NS + sid
    pltpu.sync_copy(zeros_hbm, acc.at[pl.ds(sid * RPT, RPT)])
    plsc.subcore_barrier()

    def start_gather(k, buf, sem):
        pltpu.async_copy(g_hbm.at[slabr.at[k, pl.ds(0, HB)]],
                         buf.at[pl.ds(0, HB)], sem)
        pltpu.async_copy(g_hbm.at[slabr.at[k, pl.ds(HB, CHUNK - HB)]],
                         buf.at[pl.ds(HB, CHUNK - HB)], sem)

    def wait_gather(k, buf, sem):
        pltpu.make_async_copy(g_hbm.at[slabr.at[k]], buf, sem).wait()

    def group(gi, carry):
        pltpu.sync_copy(row_hbm.at[wid, gi], slabr)
        pltpu.sync_copy(col_hbm.at[wid, gi], slabc)
        start_gather(0, buf0, sem0)

        def pair(t, c2):
            k0 = 2 * t
            start_gather(k0 + 1, buf1, sem1)
            wait_gather(k0, buf0, sem0)
            pltpu.sync_copy(buf0, acc.at[slabc.at[k0]], add=True)

            @pl.when(k0 + 2 < G)
            def _():
                start_gather(k0 + 2, buf0, sem0)

            wait_gather(k0 + 1, buf1, sem1)
            pltpu.sync_copy(buf1, acc.at[slabc.at[k0 + 1]], add=True)
            return c2

        lax.fori_loop(0, G // 2, pair, 0)
        return carry

    lax.fori_loop(0, NG, group, 0)
    plsc.subcore_barrier()
    pltpu.sync_copy(acc.at[pl.ds(sid * RPT, RPT)],
                    out_hbm.at[cid, pl.ds(sid * RPT, RPT)])


def _deg(hp):
    return 1.0 + hp[0, :, 0:1] + hp[1, :, 0:1]


def _prep_body(hist_ref, x_ref, g0_ref):
    dis = lax.rsqrt(_deg(hist_ref[...]))
    g0_ref[...] = dis * x_ref[...]


def _mid_body(hist_ref, b0_ref, g0_ref, g1_ref):
    deg = _deg(hist_ref[...])
    b0 = b0_ref[...]
    g1_ref[...] = (b0[0] + b0[1] + g0_ref[...]) / deg


def _fin_body(hist_ref, b1_ref, g1_ref, w_ref, b_ref, out_ref):
    dis = lax.rsqrt(_deg(hist_ref[...]))
    b1 = b1_ref[...]
    h2 = dis * (b1[0] + b1[1] + g1_ref[...])
    y = lax.dot_general(h2, w_ref[...], (((1,), (1,)), ((), ())),
                        preferred_element_type=jnp.float32) + b_ref[...]
    m = jnp.max(y, axis=1, keepdims=True)
    lse = m + jnp.log(jnp.sum(jnp.exp(y - m), axis=1, keepdims=True))
    out_ref[...] = y - lse


_hist_spec = pl.BlockSpec((NC, BN, FH), lambda i: (0, i, 0))
_row_spec = pl.BlockSpec((BN, F_IN), lambda i: (i, 0))
_part_spec = pl.BlockSpec((NC, BN, F_IN), lambda i: (0, i, 0))

_prep = pl.pallas_call(
    _prep_body,
    grid=(GRID,),
    in_specs=[_hist_spec, _row_spec],
    out_specs=_row_spec,
    out_shape=jax.ShapeDtypeStruct((N, F_IN), jnp.float32),
)

_mid = pl.pallas_call(
    _mid_body,
    grid=(GRID,),
    in_specs=[_hist_spec, _part_spec, _row_spec],
    out_specs=_row_spec,
    out_shape=jax.ShapeDtypeStruct((N, F_IN), jnp.float32),
)

_fin = pl.pallas_call(
    _fin_body,
    grid=(GRID,),
    in_specs=[
        _hist_spec,
        _part_spec,
        _row_spec,
        pl.BlockSpec((F_OUT, F_IN), lambda i: (0, 0)),
        pl.BlockSpec((1, F_OUT), lambda i: (0, 0)),
    ],
    out_specs=pl.BlockSpec((BN, F_OUT), lambda i: (i, 0)),
    out_shape=jax.ShapeDtypeStruct((N, F_OUT), jnp.float32),
)


def kernel(x, edge_index, W, b):
    col3 = edge_index[1].reshape(NW, NCHUNK, CHUNK)
    row4 = edge_index[0].reshape(NW, NG, G, CHUNK)
    col4 = edge_index[1].reshape(NW, NG, G, CHUNK)
    zeros_f = jnp.zeros((RPT, F_IN), jnp.float32)
    zeros_h = jnp.zeros((RPT, FH), jnp.float32)
    ones_h = jnp.ones((CHUNK, FH), jnp.float32)

    hist = _hist(col3, ones_h, zeros_h)
    g0 = _prep(hist, x)
    b0 = _prop(g0, row4, col4, zeros_f)
    g1 = _mid(hist, b0, g0)
    b1 = _prop(g1, row4, col4, zeros_f)
    return _fin(hist, b1, g1, W, b.reshape(1, F_OUT))


# speedup vs baseline: 1.0517x; 1.0517x over previous
import functools

import jax
import jax.numpy as jnp
from jax import lax
from jax.experimental import pallas as pl
from jax.experimental.pallas import tpu as pltpu
from jax.experimental.pallas import tpu_sc as plsc

N = 10000
NP = 10240
E = 320000
F_IN = 128
F_OUT = 64

NC = 2
NS = 16
NW = NC * NS
EPT = E // NW
CHUNK = 125
NCHUNK = EPT // CHUNK
G = 10
NG = NCHUNK // G
HB = 64
RPT = NP // NS
FH = 16

_MESH = plsc.VectorSubcoreMesh(core_axis_name="c", subcore_axis_name="s")

BN = 1000
GRID = N // BN


@functools.partial(
    pl.kernel,
    out_type=jax.ShapeDtypeStruct((NC, NP, FH), jnp.float32),
    mesh=_MESH,
    scratch_types=[
        pltpu.VMEM((NCHUNK, CHUNK), jnp.int32),
        pltpu.VMEM((CHUNK, FH), jnp.float32),
        pltpu.VMEM_SHARED((NP, FH), jnp.float32),
    ],
    compiler_params=pltpu.CompilerParams(use_tc_tiling_on_sc=False),
)
def _hist(col_hbm, ones_hbm, zeros_hbm, out_hbm, col_v, ones_v, acc):
    cid = lax.axis_index("c")
    sid = lax.axis_index("s")
    wid = cid * NS + sid
    pltpu.sync_copy(col_hbm.at[wid], col_v)
    pltpu.sync_copy(ones_hbm, ones_v)
    pltpu.sync_copy(zeros_hbm, acc.at[pl.ds(sid * RPT, RPT)])
    plsc.subcore_barrier()

    def body(j, carry):
        pltpu.sync_copy(ones_v, acc.at[col_v.at[j]], add=True)
        return carry

    lax.fori_loop(0, NCHUNK, body, 0)
    plsc.subcore_barrier()
    pltpu.sync_copy(acc.at[pl.ds(sid * RPT, RPT)],
                    out_hbm.at[cid, pl.ds(sid * RPT, RPT)])


@functools.partial(
    pl.kernel,
    out_type=jax.ShapeDtypeStruct((NC, NP, F_IN), jnp.float32),
    mesh=_MESH,
    scratch_types=[
        pltpu.VMEM((2 * G, CHUNK), jnp.int32),
        pltpu.VMEM((2 * G, CHUNK), jnp.int32),
        pltpu.VMEM((CHUNK, F_IN), jnp.float32),
        pltpu.VMEM((CHUNK, F_IN), jnp.float32),
        pltpu.VMEM_SHARED((NP, F_IN), jnp.float32),
        pltpu.SemaphoreType.DMA,
        pltpu.SemaphoreType.DMA,
        pltpu.SemaphoreType.DMA,
        pltpu.SemaphoreType.DMA,
    ],
)
def _prop(g_hbm, ei_hbm, zeros_hbm, out_hbm,
          slab0, slab1, buf0, buf1, acc, sem0, sem1, ssem0, ssem1):
    cid = lax.axis_index("c")
    sid = lax.axis_index("s")
    wid = cid * NS + sid
    pltpu.async_copy(ei_hbm.at[wid, 0], slab0, ssem0)
    pltpu.sync_copy(zeros_hbm, acc.at[pl.ds(sid * RPT, RPT)])
    plsc.subcore_barrier()

    def start_gather(slab, k2, buf, sem):
        pltpu.async_copy(g_hbm.at[slab.at[k2, pl.ds(0, HB)]],
                         buf.at[pl.ds(0, HB)], sem)
        pltpu.async_copy(g_hbm.at[slab.at[k2, pl.ds(HB, CHUNK - HB)]],
                         buf.at[pl.ds(HB, CHUNK - HB)], sem)

    def wait_gather(slab, k2, buf, sem):
        pltpu.make_async_copy(g_hbm.at[slab.at[k2]], buf, sem).wait()

    def run_group(slab):
        def pair(t, c2):
            k0 = 2 * t
            start_gather(slab, 2 * k0 + 2, buf1, sem1)
            wait_gather(slab, 2 * k0, buf0, sem0)
            pltpu.sync_copy(buf0, acc.at[slab.at[2 * k0 + 1]], add=True)

            @pl.when(k0 + 2 < G)
            def _():
                start_gather(slab, 2 * k0 + 4, buf0, sem0)

            wait_gather(slab, 2 * k0 + 2, buf1, sem1)
            pltpu.sync_copy(buf1, acc.at[slab.at[2 * k0 + 3]], add=True)
            return c2

        lax.fori_loop(0, G // 2, pair, 0)

    def grouppair(gp, carry):
        gi0 = 2 * gp
        pltpu.make_async_copy(ei_hbm.at[wid, gi0], slab0, ssem0).wait()
        pltpu.async_copy(ei_hbm.at[wid, gi0 + 1], slab1, ssem1)
        start_gather(slab0, 0, buf0, sem0)
        run_group(slab0)
        pltpu.make_async_copy(ei_hbm.at[wid, gi0 + 1], slab1, ssem1).wait()

        @pl.when(gi0 + 2 < NG)
        def _():
            pltpu.async_copy(ei_hbm.at[wid, gi0 + 2], slab0, ssem0)

        start_gather(slab1, 0, buf0, sem0)
        run_group(slab1)
        return carry

    lax.fori_loop(0, NG // 2, grouppair, 0)
    plsc.subcore_barrier()
    pltpu.sync_copy(acc.at[pl.ds(sid * RPT, RPT)],
                    out_hbm.at[cid, pl.ds(sid * RPT, RPT)])


def _deg(hp):
    return 1.0 + hp[0, :, 0:1] + hp[1, :, 0:1]


def _prep_body(hist_ref, x_ref, g0_ref):
    dis = lax.rsqrt(_deg(hist_ref[...]))
    g0_ref[...] = dis * x_ref[...]


def _mid_body(hist_ref, b0_ref, g0_ref, g1_ref):
    deg = _deg(hist_ref[...])
    b0 = b0_ref[...]
    g1_ref[...] = (b0[0] + b0[1] + g0_ref[...]) / deg


def _fin_body(hist_ref, b1_ref, g1_ref, w_ref, b_ref, out_ref):
    dis = lax.rsqrt(_deg(hist_ref[...]))
    b1 = b1_ref[...]
    h2 = dis * (b1[0] + b1[1] + g1_ref[...])
    y = lax.dot_general(h2, w_ref[...], (((1,), (1,)), ((), ())),
                        preferred_element_type=jnp.float32) + b_ref[...]
    m = jnp.max(y, axis=1, keepdims=True)
    lse = m + jnp.log(jnp.sum(jnp.exp(y - m), axis=1, keepdims=True))
    out_ref[...] = y - lse


_hist_spec = pl.BlockSpec((NC, BN, FH), lambda i: (0, i, 0))
_row_spec = pl.BlockSpec((BN, F_IN), lambda i: (i, 0))
_part_spec = pl.BlockSpec((NC, BN, F_IN), lambda i: (0, i, 0))

_prep = pl.pallas_call(
    _prep_body,
    grid=(GRID,),
    in_specs=[_hist_spec, _row_spec],
    out_specs=_row_spec,
    out_shape=jax.ShapeDtypeStruct((N, F_IN), jnp.float32),
)

_mid = pl.pallas_call(
    _mid_body,
    grid=(GRID,),
    in_specs=[_hist_spec, _part_spec, _row_spec],
    out_specs=_row_spec,
    out_shape=jax.ShapeDtypeStruct((N, F_IN), jnp.float32),
)

_fin = pl.pallas_call(
    _fin_body,
    grid=(GRID,),
    in_specs=[
        _hist_spec,
        _part_spec,
        _row_spec,
        pl.BlockSpec((F_OUT, F_IN), lambda i: (0, 0)),
        pl.BlockSpec((1, F_OUT), lambda i: (0, 0)),
    ],
    out_specs=pl.BlockSpec((BN, F_OUT), lambda i: (i, 0)),
    out_shape=jax.ShapeDtypeStruct((N, F_OUT), jnp.float32),
)


def kernel(x, edge_index, W, b):
    col3 = edge_index[1].reshape(NW, NCHUNK, CHUNK)
    row4 = edge_index[0].reshape(NW, NG, G, CHUNK)
    col4 = edge_index[1].reshape(NW, NG, G, CHUNK)
    ei4 = jnp.stack([row4, col4], axis=3).reshape(NW, NG, 2 * G, CHUNK)
    zeros_f = jnp.zeros((RPT, F_IN), jnp.float32)
    zeros_h = jnp.zeros((RPT, FH), jnp.float32)
    ones_h = jnp.ones((CHUNK, FH), jnp.float32)

    hist = _hist(col3, ones_h, zeros_h)
    g0 = _prep(hist, x)
    b0 = _prop(g0, ei4, zeros_f)
    g1 = _mid(hist, b0, g0)
    b1 = _prop(g1, ei4, zeros_f)
    return _fin(hist, b1, g1, W, b.reshape(1, F_OUT))


# speedup vs baseline: 1.0873x; 1.0339x over previous
import functools

import jax
import jax.numpy as jnp
from jax import lax
from jax.experimental import pallas as pl
from jax.experimental.pallas import tpu as pltpu
from jax.experimental.pallas import tpu_sc as plsc

N = 10000
NP = 10240
E = 320000
F_IN = 128
F_OUT = 64

NC = 2
NS = 16
NW = NC * NS
EPT = E // NW
CHUNK = 125
NCHUNK = EPT // CHUNK
G = 20
NG = NCHUNK // G
HB = 64
RPT = NP // NS
FH = 16

_MESH = plsc.VectorSubcoreMesh(core_axis_name="c", subcore_axis_name="s")

BN = 1000
GRID = N // BN


@functools.partial(
    pl.kernel,
    out_type=jax.ShapeDtypeStruct((NC, NP, FH), jnp.float32),
    mesh=_MESH,
    scratch_types=[
        pltpu.VMEM((NCHUNK, CHUNK), jnp.int32),
        pltpu.VMEM((CHUNK, FH), jnp.float32),
        pltpu.VMEM_SHARED((NP, FH), jnp.float32),
    ],
    compiler_params=pltpu.CompilerParams(use_tc_tiling_on_sc=False),
)
def _hist(col_hbm, ones_hbm, zeros_hbm, out_hbm, col_v, ones_v, acc):
    cid = lax.axis_index("c")
    sid = lax.axis_index("s")
    wid = cid * NS + sid
    pltpu.sync_copy(col_hbm.at[wid], col_v)
    pltpu.sync_copy(ones_hbm, ones_v)
    pltpu.sync_copy(zeros_hbm, acc.at[pl.ds(sid * RPT, RPT)])
    plsc.subcore_barrier()

    def body(j, carry):
        pltpu.sync_copy(ones_v, acc.at[col_v.at[j]], add=True)
        return carry

    lax.fori_loop(0, NCHUNK, body, 0)
    plsc.subcore_barrier()
    pltpu.sync_copy(acc.at[pl.ds(sid * RPT, RPT)],
                    out_hbm.at[cid, pl.ds(sid * RPT, RPT)])


@functools.partial(
    pl.kernel,
    out_type=jax.ShapeDtypeStruct((NC, NP, F_IN), jnp.float32),
    mesh=_MESH,
    scratch_types=[
        pltpu.VMEM((2 * G, CHUNK), jnp.int32),
        pltpu.VMEM((2 * G, CHUNK), jnp.int32),
        pltpu.VMEM((CHUNK, F_IN), jnp.float32),
        pltpu.VMEM((CHUNK, F_IN), jnp.float32),
        pltpu.VMEM_SHARED((NP, F_IN), jnp.float32),
        pltpu.SemaphoreType.DMA,
        pltpu.SemaphoreType.DMA,
        pltpu.SemaphoreType.DMA,
        pltpu.SemaphoreType.DMA,
    ],
)
def _prop(g_hbm, ei_hbm, zeros_hbm, out_hbm,
          slab0, slab1, buf0, buf1, acc, sem0, sem1, ssem0, ssem1):
    cid = lax.axis_index("c")
    sid = lax.axis_index("s")
    wid = cid * NS + sid
    pltpu.async_copy(ei_hbm.at[wid, 0], slab0, ssem0)
    pltpu.sync_copy(zeros_hbm, acc.at[pl.ds(sid * RPT, RPT)])
    plsc.subcore_barrier()

    def start_gather(slab, k2, buf, sem):
        pltpu.async_copy(g_hbm.at[slab.at[k2, pl.ds(0, HB)]],
                         buf.at[pl.ds(0, HB)], sem)
        pltpu.async_copy(g_hbm.at[slab.at[k2, pl.ds(HB, CHUNK - HB)]],
                         buf.at[pl.ds(HB, CHUNK - HB)], sem)

    def wait_gather(slab, k2, buf, sem):
        pltpu.make_async_copy(g_hbm.at[slab.at[k2]], buf, sem).wait()

    def run_group(slab):
        def pair(t, c2):
            k0 = 2 * t
            start_gather(slab, 2 * k0 + 2, buf1, sem1)
            wait_gather(slab, 2 * k0, buf0, sem0)
            pltpu.sync_copy(buf0, acc.at[slab.at[2 * k0 + 1]], add=True)

            @pl.when(k0 + 2 < G)
            def _():
                start_gather(slab, 2 * k0 + 4, buf0, sem0)

            wait_gather(slab, 2 * k0 + 2, buf1, sem1)
            pltpu.sync_copy(buf1, acc.at[slab.at[2 * k0 + 3]], add=True)
            return c2

        lax.fori_loop(0, G // 2, pair, 0)

    def grouppair(gp, carry):
        gi0 = 2 * gp
        pltpu.make_async_copy(ei_hbm.at[wid, gi0], slab0, ssem0).wait()
        pltpu.async_copy(ei_hbm.at[wid, gi0 + 1], slab1, ssem1)
        start_gather(slab0, 0, buf0, sem0)
        run_group(slab0)
        pltpu.make_async_copy(ei_hbm.at[wid, gi0 + 1], slab1, ssem1).wait()

        @pl.when(gi0 + 2 < NG)
        def _():
            pltpu.async_copy(ei_hbm.at[wid, gi0 + 2], slab0, ssem0)

        start_gather(slab1, 0, buf0, sem0)
        run_group(slab1)
        return carry

    lax.fori_loop(0, NG // 2, grouppair, 0)
    plsc.subcore_barrier()
    pltpu.sync_copy(acc.at[pl.ds(sid * RPT, RPT)],
                    out_hbm.at[cid, pl.ds(sid * RPT, RPT)])


def _deg(hp):
    return 1.0 + hp[0, :, 0:1] + hp[1, :, 0:1]


def _prep_body(hist_ref, x_ref, g0_ref):
    dis = lax.rsqrt(_deg(hist_ref[...]))
    g0_ref[...] = dis * x_ref[...]


def _mid_body(hist_ref, b0_ref, g0_ref, g1_ref):
    deg = _deg(hist_ref[...])
    b0 = b0_ref[...]
    g1_ref[...] = (b0[0] + b0[1] + g0_ref[...]) / deg


def _fin_body(hist_ref, b1_ref, g1_ref, w_ref, b_ref, out_ref):
    dis = lax.rsqrt(_deg(hist_ref[...]))
    b1 = b1_ref[...]
    h2 = dis * (b1[0] + b1[1] + g1_ref[...])
    y = lax.dot_general(h2, w_ref[...], (((1,), (1,)), ((), ())),
                        preferred_element_type=jnp.float32) + b_ref[...]
    m = jnp.max(y, axis=1, keepdims=True)
    lse = m + jnp.log(jnp.sum(jnp.exp(y - m), axis=1, keepdims=True))
    out_ref[...] = y - lse


_hist_spec = pl.BlockSpec((NC, BN, FH), lambda i: (0, i, 0))
_row_spec = pl.BlockSpec((BN, F_IN), lambda i: (i, 0))
_part_spec = pl.BlockSpec((NC, BN, F_IN), lambda i: (0, i, 0))

_prep = pl.pallas_call(
    _prep_body,
    grid=(GRID,),
    in_specs=[_hist_spec, _row_spec],
    out_specs=_row_spec,
    out_shape=jax.ShapeDtypeStruct((N, F_IN), jnp.float32),
)

_mid = pl.pallas_call(
    _mid_body,
    grid=(GRID,),
    in_specs=[_hist_spec, _part_spec, _row_spec],
    out_specs=_row_spec,
    out_shape=jax.ShapeDtypeStruct((N, F_IN), jnp.float32),
)

_fin = pl.pallas_call(
    _fin_body,
    grid=(GRID,),
    in_specs=[
        _hist_spec,
        _part_spec,
        _row_spec,
        pl.BlockSpec((F_OUT, F_IN), lambda i: (0, 0)),
        pl.BlockSpec((1, F_OUT), lambda i: (0, 0)),
    ],
    out_specs=pl.BlockSpec((BN, F_OUT), lambda i: (i, 0)),
    out_shape=jax.ShapeDtypeStruct((N, F_OUT), jnp.float32),
)


def kernel(x, edge_index, W, b):
    col3 = edge_index[1].reshape(NW, NCHUNK, CHUNK)
    row4 = edge_index[0].reshape(NW, NG, G, CHUNK)
    col4 = edge_index[1].reshape(NW, NG, G, CHUNK)
    ei4 = jnp.stack([row4, col4], axis=3).reshape(NW, NG, 2 * G, CHUNK)
    zeros_f = jnp.zeros((RPT, F_IN), jnp.float32)
    zeros_h = jnp.zeros((RPT, FH), jnp.float32)
    ones_h = jnp.ones((CHUNK, FH), jnp.float32)

    hist = _hist(col3, ones_h, zeros_h)
    g0 = _prep(hist, x)
    b0 = _prop(g0, ei4, zeros_f)
    g1 = _mid(hist, b0, g0)
    b1 = _prop(g1, ei4, zeros_f)
    return _fin(hist, b1, g1, W, b.reshape(1, F_OUT))


# speedup vs baseline: 1.0876x; 1.0003x over previous
import functools

import jax
import jax.numpy as jnp
from jax import lax
from jax.experimental import pallas as pl
from jax.experimental.pallas import tpu as pltpu
from jax.experimental.pallas import tpu_sc as plsc

N = 10000
NP = 10240
E = 320000
F_IN = 128
F_OUT = 64

NC = 2
NS = 16
NW = NC * NS
EPT = E // NW
CHUNK = 125
NCHUNK = EPT // CHUNK
G = 20
NG = NCHUNK // G
HB = 64
RPT = NP // NS
FH = 16

_MESH = plsc.VectorSubcoreMesh(core_axis_name="c", subcore_axis_name="s")

BN = 1000
GRID = N // BN


@functools.partial(
    pl.kernel,
    out_type=jax.ShapeDtypeStruct((NC, NP, FH), jnp.float32),
    mesh=_MESH,
    scratch_types=[
        pltpu.VMEM((NCHUNK, CHUNK), jnp.int32),
        pltpu.VMEM((CHUNK, FH), jnp.float32),
        pltpu.VMEM_SHARED((NP, FH), jnp.float32),
    ],
    compiler_params=pltpu.CompilerParams(use_tc_tiling_on_sc=False),
)
def _hist(col_hbm, ones_hbm, zeros_hbm, out_hbm, col_v, ones_v, acc):
    cid = lax.axis_index("c")
    sid = lax.axis_index("s")
    wid = cid * NS + sid
    pltpu.sync_copy(col_hbm.at[wid], col_v)
    pltpu.sync_copy(ones_hbm, ones_v)
    pltpu.sync_copy(zeros_hbm, acc.at[pl.ds(sid * RPT, RPT)])
    plsc.subcore_barrier()

    def body(j, carry):
        pltpu.sync_copy(ones_v, acc.at[col_v.at[j]], add=True)
        return carry

    lax.fori_loop(0, NCHUNK, body, 0)
    plsc.subcore_barrier()
    pltpu.sync_copy(acc.at[pl.ds(sid * RPT, RPT)],
                    out_hbm.at[cid, pl.ds(sid * RPT, RPT)])


@functools.partial(
    pl.kernel,
    out_type=jax.ShapeDtypeStruct((NC, NP, F_IN), jnp.float32),
    mesh=_MESH,
    scratch_types=[
        pltpu.VMEM((2 * G, CHUNK), jnp.int32),
        pltpu.VMEM((2 * G, CHUNK), jnp.int32),
        pltpu.VMEM((CHUNK, F_IN), jnp.float32),
        pltpu.VMEM((CHUNK, F_IN), jnp.float32),
        pltpu.VMEM_SHARED((NP, F_IN), jnp.float32),
        pltpu.SemaphoreType.DMA,
        pltpu.SemaphoreType.DMA,
        pltpu.SemaphoreType.DMA,
        pltpu.SemaphoreType.DMA,
    ],
)
def _prop(g_hbm, ei_hbm, zeros_hbm, out_hbm,
          slab0, slab1, buf0, buf1, acc, sem0, sem1, ssem0, ssem1):
    cid = lax.axis_index("c")
    sid = lax.axis_index("s")
    wid = cid * NS + sid
    pltpu.async_copy(ei_hbm.at[wid, 0], slab0, ssem0)
    pltpu.sync_copy(zeros_hbm, acc.at[pl.ds(sid * RPT, RPT)])
    plsc.subcore_barrier()

    def start_gather(slab, k2, buf, sem):
        pltpu.async_copy(g_hbm.at[slab.at[k2]], buf, sem)

    def wait_gather(slab, k2, buf, sem):
        pltpu.make_async_copy(g_hbm.at[slab.at[k2]], buf, sem).wait()

    def run_group(slab):
        def pair(t, c2):
            k0 = 2 * t
            start_gather(slab, 2 * k0 + 2, buf1, sem1)
            wait_gather(slab, 2 * k0, buf0, sem0)
            pltpu.sync_copy(buf0, acc.at[slab.at[2 * k0 + 1]], add=True)

            @pl.when(k0 + 2 < G)
            def _():
                start_gather(slab, 2 * k0 + 4, buf0, sem0)

            wait_gather(slab, 2 * k0 + 2, buf1, sem1)
            pltpu.sync_copy(buf1, acc.at[slab.at[2 * k0 + 3]], add=True)
            return c2

        lax.fori_loop(0, G // 2, pair, 0)

    def grouppair(gp, carry):
        gi0 = 2 * gp
        pltpu.make_async_copy(ei_hbm.at[wid, gi0], slab0, ssem0).wait()
        pltpu.async_copy(ei_hbm.at[wid, gi0 + 1], slab1, ssem1)
        start_gather(slab0, 0, buf0, sem0)
        run_group(slab0)
        pltpu.make_async_copy(ei_hbm.at[wid, gi0 + 1], slab1, ssem1).wait()

        @pl.when(gi0 + 2 < NG)
        def _():
            pltpu.async_copy(ei_hbm.at[wid, gi0 + 2], slab0, ssem0)

        start_gather(slab1, 0, buf0, sem0)
        run_group(slab1)
        return carry

    lax.fori_loop(0, NG // 2, grouppair, 0)
    plsc.subcore_barrier()
    pltpu.sync_copy(acc.at[pl.ds(sid * RPT, RPT)],
                    out_hbm.at[cid, pl.ds(sid * RPT, RPT)])


def _deg(hp):
    return 1.0 + hp[0, :, 0:1] + hp[1, :, 0:1]


def _prep_body(hist_ref, x_ref, g0_ref):
    dis = lax.rsqrt(_deg(hist_ref[...]))
    g0_ref[...] = dis * x_ref[...]


def _mid_body(hist_ref, b0_ref, g0_ref, g1_ref):
    deg = _deg(hist_ref[...])
    b0 = b0_ref[...]
    g1_ref[...] = (b0[0] + b0[1] + g0_ref[...]) / deg


def _fin_body(hist_ref, b1_ref, g1_ref, w_ref, b_ref, out_ref):
    dis = lax.rsqrt(_deg(hist_ref[...]))
    b1 = b1_ref[...]
    h2 = dis * (b1[0] + b1[1] + g1_ref[...])
    y = lax.dot_general(h2, w_ref[...], (((1,), (1,)), ((), ())),
                        preferred_element_type=jnp.float32) + b_ref[...]
    m = jnp.max(y, axis=1, keepdims=True)
    lse = m + jnp.log(jnp.sum(jnp.exp(y - m), axis=1, keepdims=True))
    out_ref[...] = y - lse


_hist_spec = pl.BlockSpec((NC, BN, FH), lambda i: (0, i, 0))
_row_spec = pl.BlockSpec((BN, F_IN), lambda i: (i, 0))
_part_spec = pl.BlockSpec((NC, BN, F_IN), lambda i: (0, i, 0))

_prep = pl.pallas_call(
    _prep_body,
    grid=(GRID,),
    in_specs=[_hist_spec, _row_spec],
    out_specs=_row_spec,
    out_shape=jax.ShapeDtypeStruct((N, F_IN), jnp.float32),
)

_mid = pl.pallas_call(
    _mid_body,
    grid=(GRID,),
    in_specs=[_hist_spec, _part_spec, _row_spec],
    out_specs=_row_spec,
    out_shape=jax.ShapeDtypeStruct((N, F_IN), jnp.float32),
)

_fin = pl.pallas_call(
    _fin_body,
    grid=(GRID,),
    in_specs=[
        _hist_spec,
        _part_spec,
        _row_spec,
        pl.BlockSpec((F_OUT, F_IN), lambda i: (0, 0)),
        pl.BlockSpec((1, F_OUT), lambda i: (0, 0)),
    ],
    out_specs=pl.BlockSpec((BN, F_OUT), lambda i: (i, 0)),
    out_shape=jax.ShapeDtypeStruct((N, F_OUT), jnp.float32),
)


def kernel(x, edge_index, W, b):
    col3 = edge_index[1].reshape(NW, NCHUNK, CHUNK)
    row4 = edge_index[0].reshape(NW, NG, G, CHUNK)
    col4 = edge_index[1].reshape(NW, NG, G, CHUNK)
    ei4 = jnp.stack([row4, col4], axis=3).reshape(NW, NG, 2 * G, CHUNK)
    zeros_f = jnp.zeros((RPT, F_IN), jnp.float32)
    zeros_h = jnp.zeros((RPT, FH), jnp.float32)
    ones_h = jnp.ones((CHUNK, FH), jnp.float32)

    hist = _hist(col3, ones_h, zeros_h)
    g0 = _prep(hist, x)
    b0 = _prop(g0, ei4, zeros_f)
    g1 = _mid(hist, b0, g0)
    b1 = _prop(g1, ei4, zeros_f)
    return _fin(hist, b1, g1, W, b.reshape(1, F_OUT))
